# baseline (speedup 1.0000x reference)
import functools

import jax
import jax.numpy as jnp
from jax import lax
from jax.experimental import pallas as pl
from jax.experimental.pallas import tpu as pltpu
from jax.experimental.pallas import tpu_sc as plsc

_B, _R, _N = 4, 4096, 4096
_STRIDE = 32
_K = _N // _STRIDE
_TOTAL = _B * _R * _K

_S_TC = 3072
_RBLK = 256

_R8_TC = _S_TC // 8
_SBPB = 512 - _R8_TC
_NW = 32
_WPB = _NW // _B
_SBPW = _SBPB // _WPB
_OPW = _SBPW * 1024
_C = 2048
_CHUNKS = _OPW // _C

_mesh = plsc.VectorSubcoreMesh(core_axis_name="c", subcore_axis_name="s")


@functools.partial(
    pl.kernel,
    out_type=jax.ShapeDtypeStruct((_B * _SBPB * 1024,), jnp.float32),
    mesh=_mesh,
    scratch_types=[
        pltpu.VMEM((2, _C, 16), jnp.float32),
        pltpu.VMEM((2, _C), jnp.float32),
        pltpu.SemaphoreType.DMA,
        pltpu.SemaphoreType.DMA,
        pltpu.SemaphoreType.DMA,
        pltpu.SemaphoreType.DMA,
    ],
    compiler_params=pltpu.CompilerParams(
        use_tc_tiling_on_sc=False, needs_layout_passes=False),
    cost_estimate=pl.CostEstimate(
        flops=0, bytes_accessed=_B * _SBPB * 1024 * 4 * 17, transcendentals=0),
)
def _select_sc(x_hbm, out_hbm, buf_v, out_v, in0, in1, ot0, ot1):
    wid = lax.axis_index("c") * 16 + lax.axis_index("s")
    b = wid // _WPB
    k = wid % _WPB
    in_base = (b * 512 + _R8_TC + k * _SBPW) * 1024
    out_base = (b * _SBPB + k * _SBPW) * 1024
    lanes = lax.iota(jnp.int32, 16)
    pperm = (lanes // 4) * 32 + (lanes % 4)
    zeros = jnp.zeros((16,), jnp.int32)
    in_sems = (in0, in1)
    out_sems = (ot0, ot1)

    def start_in(i):
        return pltpu.async_copy(
            x_hbm.at[pl.ds(in_base + i * _C, _C), 0], buf_v.at[i % 2],
            in_sems[i % 2])

    def start_out(i):
        return pltpu.async_copy(
            out_v.at[i % 2], out_hbm.at[pl.ds(out_base + i * _C, _C)],
            out_sems[i % 2])

    in_flight = {0: start_in(0)}
    out_flight = {}
    for i in range(_CHUNKS):
        if i + 1 < _CHUNKS:
            in_flight[i + 1] = start_in(i + 1)
        in_flight.pop(i).wait()

        def compact(m, carry):
            gbase = (m // 64) * 1024 + (m % 8) * 128 + ((m // 8) % 8) * 4
            out_v[i % 2, pl.ds(m * 16, 16)] = plsc.load_gather(
                buf_v, [jnp.full((16,), i % 2, jnp.int32), gbase + pperm,
                        zeros])
            return carry

        if i - 2 in out_flight:
            out_flight.pop(i - 2).wait()
        lax.fori_loop(0, _C // 16, compact, 0, unroll=8)
        out_flight[i] = start_out(i)
    for h in out_flight.values():
        h.wait()


def _tc_body(x_ref, s_ref, o_ref):
    o_ref[0] = jax.lax.dot_general(
        x_ref[0], s_ref[...], (((1,), (0,)), ((), ())),
        preferred_element_type=jnp.float32)


def _select_tc(x, sel, rows, row_off):
    noff = row_off // _RBLK
    return pl.pallas_call(
        _tc_body,
        grid=(_B, rows // _RBLK),
        in_specs=[
            pl.BlockSpec((1, _RBLK, _N), lambda b, i: (b, i + noff, 0)),
            pl.BlockSpec((_N, _K), lambda b, i: (0, 0)),
        ],
        out_specs=pl.BlockSpec((1, _RBLK, _K), lambda b, i: (b, i, 0)),
        out_shape=jax.ShapeDtypeStruct((_B, rows, _K), jnp.float32),
    )(x, sel)


def kernel(x):
    sel = jnp.zeros((_N, _K), jnp.float32).at[
        jnp.arange(0, _N, _STRIDE), jnp.arange(_K)].set(1.0)
    xv = (x.reshape(_B, 512, 8, 32, 128).transpose(0, 1, 3, 2, 4)
          .reshape(_TOTAL, 2, 16))
    out_sc = _select_sc(xv)
    out_sc = out_sc.reshape(_B, _SBPB * 8, _K)
    half = _S_TC // 2
    out_tc0 = _select_tc(x, sel, half, 0)
    out_tc1 = _select_tc(x, sel, half, half)
    return jnp.concatenate([out_tc0, out_tc1, out_sc], axis=1)


# speedup vs baseline: 1.0712x; 1.0712x over previous
import functools

import jax
import jax.numpy as jnp
from jax import lax
from jax.experimental import pallas as pl
from jax.experimental.pallas import tpu as pltpu
from jax.experimental.pallas import tpu_sc as plsc

_B, _R, _N = 4, 4096, 4096
_STRIDE = 32
_K = _N // _STRIDE
_TOTAL = _B * _R * _K

_S_TC = 3584
_RBLK = 512

_R8_TC = _S_TC // 8
_SBPB = 512 - _R8_TC
_NW = 32
_WPB = _NW // _B
_SBPW = _SBPB // _WPB
_OPW = _SBPW * 1024
_C = 2048
_CHUNKS = _OPW // _C

_mesh = plsc.VectorSubcoreMesh(core_axis_name="c", subcore_axis_name="s")


@functools.partial(
    pl.kernel,
    out_type=jax.ShapeDtypeStruct((_B * _SBPB * 1024,), jnp.float32),
    mesh=_mesh,
    scratch_types=[
        pltpu.VMEM((2, _C, 16), jnp.float32),
        pltpu.VMEM((2, _C), jnp.float32),
        pltpu.SemaphoreType.DMA,
        pltpu.SemaphoreType.DMA,
        pltpu.SemaphoreType.DMA,
        pltpu.SemaphoreType.DMA,
    ],
    compiler_params=pltpu.CompilerParams(
        use_tc_tiling_on_sc=False, needs_layout_passes=False),
    cost_estimate=pl.CostEstimate(
        flops=0, bytes_accessed=_B * _SBPB * 1024 * 4 * 17, transcendentals=0),
)
def _select_sc(x_hbm, out_hbm, buf_v, out_v, in0, in1, ot0, ot1):
    wid = lax.axis_index("c") * 16 + lax.axis_index("s")
    b = wid // _WPB
    k = wid % _WPB
    in_base = (b * 512 + _R8_TC + k * _SBPW) * 1024
    out_base = (b * _SBPB + k * _SBPW) * 1024
    lanes = lax.iota(jnp.int32, 16)
    pperm = (lanes // 4) * 32 + (lanes % 4)
    zeros = jnp.zeros((16,), jnp.int32)
    in_sems = (in0, in1)
    out_sems = (ot0, ot1)

    def start_in(i):
        return pltpu.async_copy(
            x_hbm.at[pl.ds(in_base + i * _C, _C), 0], buf_v.at[i % 2],
            in_sems[i % 2])

    def start_out(i):
        return pltpu.async_copy(
            out_v.at[i % 2], out_hbm.at[pl.ds(out_base + i * _C, _C)],
            out_sems[i % 2])

    in_flight = {0: start_in(0)}
    out_flight = {}
    for i in range(_CHUNKS):
        if i + 1 < _CHUNKS:
            in_flight[i + 1] = start_in(i + 1)
        in_flight.pop(i).wait()

        def compact(m, carry):
            gbase = (m // 64) * 1024 + (m % 8) * 128 + ((m // 8) % 8) * 4
            out_v[i % 2, pl.ds(m * 16, 16)] = plsc.load_gather(
                buf_v, [jnp.full((16,), i % 2, jnp.int32), gbase + pperm,
                        zeros])
            return carry

        if i - 2 in out_flight:
            out_flight.pop(i - 2).wait()
        lax.fori_loop(0, _C // 16, compact, 0, unroll=8)
        out_flight[i] = start_out(i)
    for h in out_flight.values():
        h.wait()


def _tc_body(x_ref, s_ref, o_ref):
    o_ref[0] = jax.lax.dot_general(
        x_ref[0], s_ref[...], (((1,), (0,)), ((), ())),
        preferred_element_type=jnp.float32)


def _select_tc(x, sel, rows, row_off):
    noff = row_off // _RBLK
    return pl.pallas_call(
        _tc_body,
        grid=(_B, rows // _RBLK),
        in_specs=[
            pl.BlockSpec((1, _RBLK, _N), lambda b, i: (b, i + noff, 0)),
            pl.BlockSpec((_N, _K), lambda b, i: (0, 0)),
        ],
        out_specs=pl.BlockSpec((1, _RBLK, _K), lambda b, i: (b, i, 0)),
        out_shape=jax.ShapeDtypeStruct((_B, rows, _K), jnp.float32),
    )(x, sel)


def kernel(x):
    sel = jnp.zeros((_N, _K), jnp.float32).at[
        jnp.arange(0, _N, _STRIDE), jnp.arange(_K)].set(1.0)
    xv = (x.reshape(_B, 512, 8, 32, 128).transpose(0, 1, 3, 2, 4)
          .reshape(_TOTAL, 2, 16))
    out_sc = _select_sc(xv)
    out_sc = out_sc.reshape(_B, _SBPB * 8, _K)
    out_tc = _select_tc(x, sel, _S_TC, 0)
    return jnp.concatenate([out_tc, out_sc], axis=1)


# speedup vs baseline: 1.1080x; 1.0343x over previous
import functools

import jax
import jax.numpy as jnp
from jax import lax
from jax.experimental import pallas as pl
from jax.experimental.pallas import tpu as pltpu
from jax.experimental.pallas import tpu_sc as plsc

_B, _R, _N = 4, 4096, 4096
_STRIDE = 32
_K = _N // _STRIDE
_TOTAL = _B * _R * _K

_S_TC = 3072
_RBLK = 512

_R8_TC = _S_TC // 8
_SBPB = 512 - _R8_TC
_NW = 32
_WPB = _NW // _B
_SBPW = _SBPB // _WPB
_OPW = _SBPW * 1024
_C = 2048
_CHUNKS = _OPW // _C

_mesh = plsc.VectorSubcoreMesh(core_axis_name="c", subcore_axis_name="s")


@functools.partial(
    pl.kernel,
    out_type=jax.ShapeDtypeStruct((_B * _SBPB * 1024,), jnp.float32),
    mesh=_mesh,
    scratch_types=[
        pltpu.VMEM((2, _C, 16), jnp.float32),
        pltpu.VMEM((2, _C), jnp.float32),
        pltpu.SemaphoreType.DMA,
        pltpu.SemaphoreType.DMA,
        pltpu.SemaphoreType.DMA,
        pltpu.SemaphoreType.DMA,
    ],
    compiler_params=pltpu.CompilerParams(
        use_tc_tiling_on_sc=False, needs_layout_passes=False),
    cost_estimate=pl.CostEstimate(
        flops=0, bytes_accessed=_B * _SBPB * 1024 * 4 * 17, transcendentals=0),
)
def _select_sc(x_hbm, out_hbm, buf_v, out_v, in0, in1, ot0, ot1):
    wid = lax.axis_index("c") * 16 + lax.axis_index("s")
    b = wid // _WPB
    k = wid % _WPB
    in_base = (b * 512 + _R8_TC + k * _SBPW) * 1024
    out_base = (b * _SBPB + k * _SBPW) * 1024
    lanes = lax.iota(jnp.int32, 16)
    pperm = (lanes // 4) * 32 + (lanes % 4)
    zeros = jnp.zeros((16,), jnp.int32)
    in_sems = (in0, in1)
    out_sems = (ot0, ot1)

    def start_in(i):
        return pltpu.async_copy(
            x_hbm.at[pl.ds(in_base + i * _C, _C), 0], buf_v.at[i % 2],
            in_sems[i % 2])

    def start_out(i):
        return pltpu.async_copy(
            out_v.at[i % 2], out_hbm.at[pl.ds(out_base + i * _C, _C)],
            out_sems[i % 2])

    in_flight = {0: start_in(0)}
    out_flight = {}
    for i in range(_CHUNKS):
        if i + 1 < _CHUNKS:
            in_flight[i + 1] = start_in(i + 1)
        in_flight.pop(i).wait()

        def compact(m, carry):
            gbase = (m // 64) * 1024 + (m % 8) * 128 + ((m // 8) % 8) * 4
            out_v[i % 2, pl.ds(m * 16, 16)] = plsc.load_gather(
                buf_v, [jnp.full((16,), i % 2, jnp.int32), gbase + pperm,
                        zeros])
            return carry

        if i - 2 in out_flight:
            out_flight.pop(i - 2).wait()
        lax.fori_loop(0, _C // 16, compact, 0, unroll=8)
        out_flight[i] = start_out(i)
    for h in out_flight.values():
        h.wait()


def _tc_body(x_ref, s_ref, o_ref):
    o_ref[0] = jax.lax.dot_general(
        x_ref[0], s_ref[...], (((1,), (0,)), ((), ())),
        preferred_element_type=jnp.float32)


def _select_tc(x, sel, rows, row_off):
    noff = row_off // _RBLK
    return pl.pallas_call(
        _tc_body,
        grid=(_B, rows // _RBLK),
        in_specs=[
            pl.BlockSpec((1, _RBLK, _N), lambda b, i: (b, i + noff, 0)),
            pl.BlockSpec((_N, _K), lambda b, i: (0, 0)),
        ],
        out_specs=pl.BlockSpec((1, _RBLK, _K), lambda b, i: (b, i, 0)),
        out_shape=jax.ShapeDtypeStruct((_B, rows, _K), jnp.float32),
    )(x, sel)


def kernel(x):
    sel = jnp.zeros((_N, _K), jnp.float32).at[
        jnp.arange(0, _N, _STRIDE), jnp.arange(_K)].set(1.0)
    xv = (x.reshape(_B, 512, 8, 32, 128).transpose(0, 1, 3, 2, 4)
          .reshape(_TOTAL, 2, 16))
    out_sc = _select_sc(xv)
    out_sc = out_sc.reshape(_B, _SBPB * 8, _K)
    out_tc = _select_tc(x, sel, _S_TC, 0)
    return jnp.concatenate([out_tc, out_sc], axis=1)


# speedup vs baseline: 1.1355x; 1.0248x over previous
import functools

import jax
import jax.numpy as jnp
from jax import lax
from jax.experimental import pallas as pl
from jax.experimental.pallas import tpu as pltpu
from jax.experimental.pallas import tpu_sc as plsc

_B, _R, _N = 4, 4096, 4096
_STRIDE = 32
_K = _N // _STRIDE
_TOTAL = _B * _R * _K

_S_TC = 2560
_RBLK = 512

_R8_TC = _S_TC // 8
_SBPB = 512 - _R8_TC
_NW = 32
_WPB = _NW // _B
_SBPW = _SBPB // _WPB
_OPW = _SBPW * 1024
_C = 2048
_CHUNKS = _OPW // _C

_mesh = plsc.VectorSubcoreMesh(core_axis_name="c", subcore_axis_name="s")


@functools.partial(
    pl.kernel,
    out_type=jax.ShapeDtypeStruct((_B * _SBPB * 1024,), jnp.float32),
    mesh=_mesh,
    scratch_types=[
        pltpu.VMEM((2, _C, 16), jnp.float32),
        pltpu.VMEM((2, _C), jnp.float32),
        pltpu.SemaphoreType.DMA,
        pltpu.SemaphoreType.DMA,
        pltpu.SemaphoreType.DMA,
        pltpu.SemaphoreType.DMA,
    ],
    compiler_params=pltpu.CompilerParams(
        use_tc_tiling_on_sc=False, needs_layout_passes=False),
    cost_estimate=pl.CostEstimate(
        flops=0, bytes_accessed=_B * _SBPB * 1024 * 4 * 17, transcendentals=0),
)
def _select_sc(x_hbm, out_hbm, buf_v, out_v, in0, in1, ot0, ot1):
    wid = lax.axis_index("c") * 16 + lax.axis_index("s")
    b = wid // _WPB
    k = wid % _WPB
    in_base = (b * 512 + _R8_TC + k * _SBPW) * 1024
    out_base = (b * _SBPB + k * _SBPW) * 1024
    lanes = lax.iota(jnp.int32, 16)
    pperm = (lanes // 4) * 32 + (lanes % 4)
    zeros = jnp.zeros((16,), jnp.int32)
    in_sems = (in0, in1)
    out_sems = (ot0, ot1)

    def start_in(i):
        return pltpu.async_copy(
            x_hbm.at[pl.ds(in_base + i * _C, _C), 0], buf_v.at[i % 2],
            in_sems[i % 2])

    def start_out(i):
        return pltpu.async_copy(
            out_v.at[i % 2], out_hbm.at[pl.ds(out_base + i * _C, _C)],
            out_sems[i % 2])

    in_flight = {0: start_in(0)}
    out_flight = {}
    for i in range(_CHUNKS):
        if i + 1 < _CHUNKS:
            in_flight[i + 1] = start_in(i + 1)
        in_flight.pop(i).wait()

        def compact(m, carry):
            gbase = (m // 64) * 1024 + (m % 8) * 128 + ((m // 8) % 8) * 4
            out_v[i % 2, pl.ds(m * 16, 16)] = plsc.load_gather(
                buf_v, [jnp.full((16,), i % 2, jnp.int32), gbase + pperm,
                        zeros])
            return carry

        if i - 2 in out_flight:
            out_flight.pop(i - 2).wait()
        lax.fori_loop(0, _C // 16, compact, 0, unroll=8)
        out_flight[i] = start_out(i)
    for h in out_flight.values():
        h.wait()


def _tc_body(x_ref, s_ref, o_ref):
    o_ref[0] = jax.lax.dot_general(
        x_ref[0], s_ref[...], (((1,), (0,)), ((), ())),
        preferred_element_type=jnp.float32)


def _select_tc(x, sel, rows, row_off):
    noff = row_off // _RBLK
    return pl.pallas_call(
        _tc_body,
        grid=(_B, rows // _RBLK),
        in_specs=[
            pl.BlockSpec((1, _RBLK, _N), lambda b, i: (b, i + noff, 0)),
            pl.BlockSpec((_N, _K), lambda b, i: (0, 0)),
        ],
        out_specs=pl.BlockSpec((1, _RBLK, _K), lambda b, i: (b, i, 0)),
        out_shape=jax.ShapeDtypeStruct((_B, rows, _K), jnp.float32),
    )(x, sel)


def kernel(x):
    sel = jnp.zeros((_N, _K), jnp.float32).at[
        jnp.arange(0, _N, _STRIDE), jnp.arange(_K)].set(1.0)
    xv = (x.reshape(_B, 512, 8, 32, 128).transpose(0, 1, 3, 2, 4)
          .reshape(_TOTAL, 2, 16))
    out_sc = _select_sc(xv)
    out_sc = out_sc.reshape(_B, _SBPB * 8, _K)
    out_tc = _select_tc(x, sel, _S_TC, 0)
    return jnp.concatenate([out_tc, out_sc], axis=1)


# speedup vs baseline: 1.1723x; 1.0324x over previous
import functools

import jax
import jax.numpy as jnp
from jax import lax
from jax.experimental import pallas as pl
from jax.experimental.pallas import tpu as pltpu
from jax.experimental.pallas import tpu_sc as plsc

_B, _R, _N = 4, 4096, 4096
_STRIDE = 32
_K = _N // _STRIDE
_TOTAL = _B * _R * _K

_S_TC = 2048
_RBLK = 512

_R8_TC = _S_TC // 8
_SBPB = 512 - _R8_TC
_NW = 32
_WPB = _NW // _B
_SBPW = _SBPB // _WPB
_OPW = _SBPW * 1024
_C = 2048
_CHUNKS = _OPW // _C

_mesh = plsc.VectorSubcoreMesh(core_axis_name="c", subcore_axis_name="s")


@functools.partial(
    pl.kernel,
    out_type=jax.ShapeDtypeStruct((_B * _SBPB * 1024,), jnp.float32),
    mesh=_mesh,
    scratch_types=[
        pltpu.VMEM((2, _C, 16), jnp.float32),
        pltpu.VMEM((2, _C), jnp.float32),
        pltpu.SemaphoreType.DMA,
        pltpu.SemaphoreType.DMA,
        pltpu.SemaphoreType.DMA,
        pltpu.SemaphoreType.DMA,
    ],
    compiler_params=pltpu.CompilerParams(
        use_tc_tiling_on_sc=False, needs_layout_passes=False),
    cost_estimate=pl.CostEstimate(
        flops=0, bytes_accessed=_B * _SBPB * 1024 * 4 * 17, transcendentals=0),
)
def _select_sc(x_hbm, out_hbm, buf_v, out_v, in0, in1, ot0, ot1):
    wid = lax.axis_index("c") * 16 + lax.axis_index("s")
    b = wid // _WPB
    k = wid % _WPB
    in_base = (b * 512 + _R8_TC + k * _SBPW) * 1024
    out_base = (b * _SBPB + k * _SBPW) * 1024
    lanes = lax.iota(jnp.int32, 16)
    pperm = (lanes // 4) * 32 + (lanes % 4)
    zeros = jnp.zeros((16,), jnp.int32)
    in_sems = (in0, in1)
    out_sems = (ot0, ot1)

    def start_in(i):
        return pltpu.async_copy(
            x_hbm.at[pl.ds(in_base + i * _C, _C), 0], buf_v.at[i % 2],
            in_sems[i % 2])

    def start_out(i):
        return pltpu.async_copy(
            out_v.at[i % 2], out_hbm.at[pl.ds(out_base + i * _C, _C)],
            out_sems[i % 2])

    in_flight = {0: start_in(0)}
    out_flight = {}
    for i in range(_CHUNKS):
        if i + 1 < _CHUNKS:
            in_flight[i + 1] = start_in(i + 1)
        in_flight.pop(i).wait()

        def compact(m, carry):
            gbase = (m // 64) * 1024 + (m % 8) * 128 + ((m // 8) % 8) * 4
            out_v[i % 2, pl.ds(m * 16, 16)] = plsc.load_gather(
                buf_v, [jnp.full((16,), i % 2, jnp.int32), gbase + pperm,
                        zeros])
            return carry

        if i - 2 in out_flight:
            out_flight.pop(i - 2).wait()
        lax.fori_loop(0, _C // 16, compact, 0, unroll=8)
        out_flight[i] = start_out(i)
    for h in out_flight.values():
        h.wait()


def _tc_body(x_ref, s_ref, o_ref):
    o_ref[0] = jax.lax.dot_general(
        x_ref[0], s_ref[...], (((1,), (0,)), ((), ())),
        preferred_element_type=jnp.float32)


def _select_tc(x, sel, rows, row_off):
    noff = row_off // _RBLK
    return pl.pallas_call(
        _tc_body,
        grid=(_B, rows // _RBLK),
        in_specs=[
            pl.BlockSpec((1, _RBLK, _N), lambda b, i: (b, i + noff, 0)),
            pl.BlockSpec((_N, _K), lambda b, i: (0, 0)),
        ],
        out_specs=pl.BlockSpec((1, _RBLK, _K), lambda b, i: (b, i, 0)),
        out_shape=jax.ShapeDtypeStruct((_B, rows, _K), jnp.float32),
    )(x, sel)


def kernel(x):
    sel = jnp.zeros((_N, _K), jnp.float32).at[
        jnp.arange(0, _N, _STRIDE), jnp.arange(_K)].set(1.0)
    xv = (x.reshape(_B, 512, 8, 32, 128).transpose(0, 1, 3, 2, 4)
          .reshape(_TOTAL, 2, 16))
    out_sc = _select_sc(xv)
    out_sc = out_sc.reshape(_B, _SBPB * 8, _K)
    out_tc = _select_tc(x, sel, _S_TC, 0)
    return jnp.concatenate([out_tc, out_sc], axis=1)
